# Initial kernel scaffold; baseline (speedup 1.0000x reference)
import functools

import jax
import jax.numpy as jnp
from jax import lax
from jax.experimental import pallas as pl
from jax.experimental.pallas import tpu as pltpu
from jax.experimental.pallas import tpu_sc as plsc

N = 10000
D = 128
E = 320000
NC = 2
NS = 16
NW = NC * NS
NPAD = 10240
CH = 128
EPT = 10240
NCH = EPT // CH
E_PAD = NW * EPT
RPT = NPAD // NS

_MESH = plsc.VectorSubcoreMesh(core_axis_name="c", subcore_axis_name="s")


def _deg_body(dstg_hbm, cnt_out, cnt_sh, idxd_v, ones_v, zer_v):
    c = lax.axis_index("c")
    s = lax.axis_index("s")
    w = c * NS + s
    for i in range(8):
        ones_v[pl.ds(i * 16, 16)] = jnp.ones((16,), jnp.float32)
    for i in range(RPT // 16):
        zer_v[pl.ds(i * 16, 16)] = jnp.zeros((16,), jnp.float32)
    pltpu.sync_copy(zer_v, cnt_sh.at[pl.ds(s * RPT, RPT)])
    pltpu.sync_copy(dstg_hbm.at[w], idxd_v)
    plsc.subcore_barrier()

    @pl.loop(0, NCH)
    def _chunk(g):
        pltpu.sync_copy(ones_v, cnt_sh.at[idxd_v.at[g]], add=True)

    plsc.subcore_barrier()
    pltpu.sync_copy(cnt_sh.at[pl.ds(s * RPT, RPT)],
                    cnt_out.at[c, pl.ds(s * RPT, RPT)])


_deg_kernel = pl.kernel(
    _deg_body,
    out_type=jax.ShapeDtypeStruct((NC, NPAD), jnp.float32),
    mesh=_MESH,
    scratch_types=[
        pltpu.VMEM_SHARED((NPAD,), jnp.float32),
        pltpu.VMEM((NCH, CH), jnp.int32),
        pltpu.VMEM((CH,), jnp.float32),
        pltpu.VMEM((RPT,), jnp.float32),
    ],
)


def _agg_body(hs_hbm, srcg_hbm, dstg_hbm, zeros_hbm, agg_out,
              agg_sh, idxs_v, idxd_v, rows0, rows1, sem0, sem1):
    c = lax.axis_index("c")
    s = lax.axis_index("s")
    w = c * NS + s
    pltpu.sync_copy(zeros_hbm.at[pl.ds(s * RPT, RPT)],
                    agg_sh.at[pl.ds(s * RPT, RPT)])
    pltpu.sync_copy(srcg_hbm.at[w], idxs_v)
    pltpu.sync_copy(dstg_hbm.at[w], idxd_v)
    plsc.subcore_barrier()

    pltpu.async_copy(hs_hbm.at[idxs_v.at[0]], rows0, sem0)

    @pl.loop(0, NCH, step=2)
    def _chunks(g):
        pltpu.async_copy(hs_hbm.at[idxs_v.at[g + 1]], rows1, sem1)
        pltpu.make_async_copy(hs_hbm.at[idxs_v.at[0]], rows0, sem0).wait()
        pltpu.sync_copy(rows0, agg_sh.at[idxd_v.at[g]], add=True)

        @pl.when(g + 2 < NCH)
        def _():
            pltpu.async_copy(hs_hbm.at[idxs_v.at[g + 2]], rows0, sem0)

        pltpu.make_async_copy(hs_hbm.at[idxs_v.at[0]], rows1, sem1).wait()
        pltpu.sync_copy(rows1, agg_sh.at[idxd_v.at[g + 1]], add=True)

    plsc.subcore_barrier()
    pltpu.sync_copy(agg_sh.at[pl.ds(s * RPT, RPT)],
                    agg_out.at[c, pl.ds(s * RPT, RPT)])


_agg_kernel = pl.kernel(
    _agg_body,
    out_type=jax.ShapeDtypeStruct((NC, NPAD, D), jnp.float32),
    mesh=_MESH,
    scratch_types=[
        pltpu.VMEM_SHARED((NPAD, D), jnp.float32),
        pltpu.VMEM((NCH, CH), jnp.int32),
        pltpu.VMEM((NCH, CH), jnp.int32),
        pltpu.VMEM((CH, D), jnp.float32),
        pltpu.VMEM((CH, D), jnp.float32),
        pltpu.SemaphoreType.DMA,
        pltpu.SemaphoreType.DMA,
    ],
)


_R = 512


def _dinv_of(cnt_ref):
    return lax.rsqrt(cnt_ref[0, :] + cnt_ref[1, :] + 1.0)


def _mm_first_body(x_ref, w_ref, cnt_ref, out_ref):
    dinv = _dinv_of(cnt_ref)
    h = jnp.dot(x_ref[...], w_ref[...], preferred_element_type=jnp.float32)
    out_ref[...] = h * dinv[:, None]


def _mm_mid_body(agg_ref, hs_ref, cnt_ref, w_ref, b_ref, out_ref, *, leaky):
    dinv = _dinv_of(cnt_ref)
    a = agg_ref[0] + agg_ref[1] + hs_ref[...]
    xn = dinv[:, None] * a + b_ref[...]
    if leaky:
        xn = jnp.where(xn >= 0, xn, 0.01 * xn)
    h = jnp.dot(xn, w_ref[...], preferred_element_type=jnp.float32)
    out_ref[...] = h * dinv[:, None]


def _fin_body(agg_ref, hs_ref, cnt_ref, b_ref, out_ref):
    dinv = _dinv_of(cnt_ref)
    a = agg_ref[0] + agg_ref[1] + hs_ref[...]
    xn = dinv[:, None] * a + b_ref[...]
    out_ref[...] = jnp.where(xn >= 0, xn, 0.01 * xn)


_spec_rows = pl.BlockSpec((_R, D), lambda i: (i, 0))
_spec_w = pl.BlockSpec((D, D), lambda i: (0, 0))
_spec_cnt = pl.BlockSpec((NC, _R), lambda i: (0, i))
_spec_agg = pl.BlockSpec((NC, _R, D), lambda i: (0, i, 0))
_spec_b = pl.BlockSpec((1, D), lambda i: (0, 0))
_out_rows = jax.ShapeDtypeStruct((NPAD, D), jnp.float32)

_mm_first = pl.pallas_call(
    _mm_first_body,
    grid=(NPAD // _R,),
    in_specs=[_spec_rows, _spec_w, _spec_cnt],
    out_specs=_spec_rows,
    out_shape=_out_rows,
)

_mm_mid_leaky = pl.pallas_call(
    functools.partial(_mm_mid_body, leaky=True),
    grid=(NPAD // _R,),
    in_specs=[_spec_agg, _spec_rows, _spec_cnt, _spec_w, _spec_b],
    out_specs=_spec_rows,
    out_shape=_out_rows,
)

_mm_mid_plain = pl.pallas_call(
    functools.partial(_mm_mid_body, leaky=False),
    grid=(NPAD // _R,),
    in_specs=[_spec_agg, _spec_rows, _spec_cnt, _spec_w, _spec_b],
    out_specs=_spec_rows,
    out_shape=_out_rows,
)

_fin = pl.pallas_call(
    _fin_body,
    grid=(NPAD // _R,),
    in_specs=[_spec_agg, _spec_rows, _spec_cnt, _spec_b],
    out_specs=_spec_rows,
    out_shape=_out_rows,
)


def kernel(x, edge_index, W1, b1, W2, b2, W3, b3):
    ei = edge_index.astype(jnp.int32)
    pad = jnp.full((E_PAD - E,), N, jnp.int32)
    srcg = jnp.concatenate([ei[0], pad]).reshape(NW, NCH, CH)
    dstg = jnp.concatenate([ei[1], pad]).reshape(NW, NCH, CH)
    xp = jnp.pad(x, ((0, NPAD - N), (0, 0)))
    zeros2d = jnp.zeros((NPAD, D), jnp.float32)

    cnt = _deg_kernel(dstg)

    hs = _mm_first(xp, W1, cnt)
    agg = _agg_kernel(hs, srcg, dstg, zeros2d)
    hs = _mm_mid_leaky(agg, hs, cnt, W2, b1.reshape(1, D))
    agg = _agg_kernel(hs, srcg, dstg, zeros2d)
    hs = _mm_mid_plain(agg, hs, cnt, W3, b2.reshape(1, D))
    agg = _agg_kernel(hs, srcg, dstg, zeros2d)
    out = _fin(agg, hs, cnt, b3.reshape(1, D))
    return out[:N]


# speedup vs baseline: 7.5498x; 7.5498x over previous
import functools

import jax
import jax.numpy as jnp
from jax import lax
from jax.experimental import pallas as pl
from jax.experimental.pallas import tpu as pltpu
from jax.experimental.pallas import tpu_sc as plsc

N = 10000
D = 128
E = 320000
NC = 2
NS = 16
NW = NC * NS
NPAD = 10240
CH = 128
EPT = 10240
NCH = EPT // CH
E_PAD = NW * EPT
RPT = NPAD // NS

_MESH = plsc.VectorSubcoreMesh(core_axis_name="c", subcore_axis_name="s")


def _deg_body(dstg_hbm, cnt_out, cnt_sh, idxd_v, ones_v, zer_v):
    c = lax.axis_index("c")
    s = lax.axis_index("s")
    w = c * NS + s
    for i in range(8):
        ones_v[pl.ds(i * 16, 16)] = jnp.ones((16,), jnp.float32)
    for i in range(RPT // 16):
        zer_v[pl.ds(i * 16, 16)] = jnp.zeros((16,), jnp.float32)
    pltpu.sync_copy(zer_v, cnt_sh.at[pl.ds(s * RPT, RPT)])
    pltpu.sync_copy(dstg_hbm.at[w], idxd_v)
    plsc.subcore_barrier()

    @pl.loop(0, NCH)
    def _chunk(g):
        pltpu.sync_copy(ones_v, cnt_sh.at[idxd_v.at[g]], add=True)

    plsc.subcore_barrier()
    pltpu.sync_copy(cnt_sh.at[pl.ds(s * RPT, RPT)],
                    cnt_out.at[c, pl.ds(s * RPT, RPT)])


_deg_kernel = pl.kernel(
    _deg_body,
    out_type=jax.ShapeDtypeStruct((NC, NPAD), jnp.float32),
    mesh=_MESH,
    scratch_types=[
        pltpu.VMEM_SHARED((NPAD,), jnp.float32),
        pltpu.VMEM((NCH, CH), jnp.int32),
        pltpu.VMEM((CH,), jnp.float32),
        pltpu.VMEM((RPT,), jnp.float32),
    ],
)


HALF = NCH // 2


def _agg_body(hs_hbm, srcg_hbm, dstg_hbm, agg_out,
              agg_sh, idxs_v, idxd_v, rows0, rows1, sem0, sem1):
    c = lax.axis_index("c")
    s = lax.axis_index("s")
    w = c * NS + s

    @pl.loop(0, CH)
    def _zr(i):
        @pl.loop(0, D // 16)
        def _zc(j):
            rows0[i, pl.ds(j * 16, 16)] = jnp.zeros((16,), jnp.float32)

    @pl.loop(0, RPT // CH)
    def _z(j):
        pltpu.sync_copy(rows0, agg_sh.at[pl.ds(s * RPT + j * CH, CH)])

    plsc.subcore_barrier()

    for h in range(2):
        pltpu.sync_copy(srcg_hbm.at[w, pl.ds(h * HALF, HALF)], idxs_v)
        pltpu.sync_copy(dstg_hbm.at[w, pl.ds(h * HALF, HALF)], idxd_v)
        pltpu.async_copy(hs_hbm.at[idxs_v.at[0]], rows0, sem0)

        @pl.loop(0, HALF, step=2)
        def _chunks(g):
            pltpu.async_copy(hs_hbm.at[idxs_v.at[g + 1]], rows1, sem1)
            pltpu.make_async_copy(hs_hbm.at[idxs_v.at[0]], rows0, sem0).wait()
            pltpu.sync_copy(rows0, agg_sh.at[idxd_v.at[g]], add=True)

            @pl.when(g + 2 < HALF)
            def _():
                pltpu.async_copy(hs_hbm.at[idxs_v.at[g + 2]], rows0, sem0)

            pltpu.make_async_copy(hs_hbm.at[idxs_v.at[0]], rows1, sem1).wait()
            pltpu.sync_copy(rows1, agg_sh.at[idxd_v.at[g + 1]], add=True)

    plsc.subcore_barrier()

    @pl.loop(0, RPT // CH)
    def _wb(j):
        pltpu.sync_copy(agg_sh.at[pl.ds(s * RPT + j * CH, CH)], rows0)
        pltpu.sync_copy(rows0, agg_out.at[c, pl.ds(s * RPT + j * CH, CH)])


_agg_kernel = pl.kernel(
    _agg_body,
    out_type=jax.ShapeDtypeStruct((NC, NPAD, D), jnp.float32),
    mesh=_MESH,
    scratch_types=[
        pltpu.VMEM_SHARED((NPAD, D), jnp.float32),
        pltpu.VMEM((HALF, CH), jnp.int32),
        pltpu.VMEM((HALF, CH), jnp.int32),
        pltpu.VMEM((CH, D), jnp.float32),
        pltpu.VMEM((CH, D), jnp.float32),
        pltpu.SemaphoreType.DMA,
        pltpu.SemaphoreType.DMA,
    ],
)


_R = 512


def _dinv_of(cnt_ref):
    return lax.rsqrt(cnt_ref[0, :] + cnt_ref[1, :] + 1.0)


def _mm_first_body(x_ref, w_ref, cnt_ref, out_ref):
    dinv = _dinv_of(cnt_ref)
    h = jnp.dot(x_ref[...], w_ref[...], preferred_element_type=jnp.float32)
    out_ref[...] = h * dinv[:, None]


def _mm_mid_body(agg_ref, hs_ref, cnt_ref, w_ref, b_ref, out_ref, *, leaky):
    dinv = _dinv_of(cnt_ref)
    a = agg_ref[0] + agg_ref[1] + hs_ref[...]
    xn = dinv[:, None] * a + b_ref[...]
    if leaky:
        xn = jnp.where(xn >= 0, xn, 0.01 * xn)
    h = jnp.dot(xn, w_ref[...], preferred_element_type=jnp.float32)
    out_ref[...] = h * dinv[:, None]


def _fin_body(agg_ref, hs_ref, cnt_ref, b_ref, out_ref):
    dinv = _dinv_of(cnt_ref)
    a = agg_ref[0] + agg_ref[1] + hs_ref[...]
    xn = dinv[:, None] * a + b_ref[...]
    out_ref[...] = jnp.where(xn >= 0, xn, 0.01 * xn)


_spec_rows = pl.BlockSpec((_R, D), lambda i: (i, 0))
_spec_w = pl.BlockSpec((D, D), lambda i: (0, 0))
_spec_cnt = pl.BlockSpec((NC, _R), lambda i: (0, i))
_spec_agg = pl.BlockSpec((NC, _R, D), lambda i: (0, i, 0))
_spec_b = pl.BlockSpec((1, D), lambda i: (0, 0))
_out_rows = jax.ShapeDtypeStruct((NPAD, D), jnp.float32)

_mm_first = pl.pallas_call(
    _mm_first_body,
    grid=(NPAD // _R,),
    in_specs=[_spec_rows, _spec_w, _spec_cnt],
    out_specs=_spec_rows,
    out_shape=_out_rows,
)

_mm_mid_leaky = pl.pallas_call(
    functools.partial(_mm_mid_body, leaky=True),
    grid=(NPAD // _R,),
    in_specs=[_spec_agg, _spec_rows, _spec_cnt, _spec_w, _spec_b],
    out_specs=_spec_rows,
    out_shape=_out_rows,
)

_mm_mid_plain = pl.pallas_call(
    functools.partial(_mm_mid_body, leaky=False),
    grid=(NPAD // _R,),
    in_specs=[_spec_agg, _spec_rows, _spec_cnt, _spec_w, _spec_b],
    out_specs=_spec_rows,
    out_shape=_out_rows,
)

_fin = pl.pallas_call(
    _fin_body,
    grid=(NPAD // _R,),
    in_specs=[_spec_agg, _spec_rows, _spec_cnt, _spec_b],
    out_specs=_spec_rows,
    out_shape=_out_rows,
)


def kernel(x, edge_index, W1, b1, W2, b2, W3, b3):
    ei = edge_index.astype(jnp.int32)
    pad = jnp.full((E_PAD - E,), N, jnp.int32)
    srcg = jnp.concatenate([ei[0], pad]).reshape(NW, NCH, CH)
    dstg = jnp.concatenate([ei[1], pad]).reshape(NW, NCH, CH)
    xp = jnp.pad(x, ((0, NPAD - N), (0, 0)))

    cnt = _deg_kernel(dstg)

    hs = _mm_first(xp, W1, cnt)
    agg = _agg_kernel(hs, srcg, dstg)
    hs = _mm_mid_leaky(agg, hs, cnt, W2, b1.reshape(1, D))
    agg = _agg_kernel(hs, srcg, dstg)
    hs = _mm_mid_plain(agg, hs, cnt, W3, b2.reshape(1, D))
    agg = _agg_kernel(hs, srcg, dstg)
    out = _fin(agg, hs, cnt, b3.reshape(1, D))
    return out[:N]
